# Initial kernel scaffold; baseline (speedup 1.0000x reference)
import jax
import jax.numpy as jnp
from jax.experimental import pallas as pl


def kernel(feat, edge, Wq1, bq1, Wk1, bk1, Wv1, bv1, Ws1, bs1, Wq2, bq2, Wk2, bk2, Wv2, bv2, Ws2, bs2, Wq3, bq3, Wk3, bk3, Wv3, bv3, Ws3, bs3, Wq4, bq4, Wk4, bk4, Wv4, bv4, Ws4, bs4):
    raise NotImplementedError("write your pallas kernel here")


# speedup vs baseline: 1.7219x; 1.7219x over previous
import functools
import math

import jax
import jax.numpy as jnp
from jax import lax
from jax.experimental import pallas as pl
from jax.experimental.pallas import tpu as pltpu
from jax.experimental.pallas import tpu_sc as plsc

N = 10000
E = 160000
N_PAD = 10240
C = 128
E_PAD = 163840
NCH = E_PAD // C
NC, NS, L = 2, 16, 16
NW = NC * NS
CH_A = NCH // NW
EW_A = CH_A * C
CH_B = NCH // NS
EW_B = CH_B * C
RPS = N_PAD // NS
C2 = 64
NCH2 = E_PAD // C2
CH_B2 = NCH2 // NS
G2 = C2 // L
AB = 640

_f32 = jnp.float32
_i32 = jnp.int32


def _proj_body(x_ref, w_ref, b_ref, q_ref, k_ref, v_ref, s_ref, *, fo, elu):
    x = x_ref[...]
    if elu:
        x = jnp.where(x > 0, x, jnp.exp(x) - 1.0)
    acc = jnp.dot(x, w_ref[...], preferred_element_type=_f32) + b_ref[...]
    half = fo // 2
    q_ref[...] = acc[:, :fo]
    k_ref[...] = acc[:, fo:2 * fo]
    v_ref[0] = acc[:, 2 * fo:2 * fo + half]
    v_ref[1] = acc[:, 2 * fo + half:3 * fo]
    s_ref[...] = acc[:, 3 * fo:]


def _proj(x, wcat, bcat, fo, elu):
    fi = x.shape[1]
    half = fo // 2
    bm = 1024
    grid = (N_PAD // bm,)
    return pl.pallas_call(
        functools.partial(_proj_body, fo=fo, elu=elu),
        grid=grid,
        in_specs=[
            pl.BlockSpec((bm, fi), lambda i: (i, 0)),
            pl.BlockSpec((fi, 4 * fo), lambda i: (0, 0)),
            pl.BlockSpec((1, 4 * fo), lambda i: (0, 0)),
        ],
        out_specs=[
            pl.BlockSpec((bm, fo), lambda i: (i, 0)),
            pl.BlockSpec((bm, fo), lambda i: (i, 0)),
            pl.BlockSpec((2, bm, half), lambda i: (0, i, 0)),
            pl.BlockSpec((bm, fo), lambda i: (i, 0)),
        ],
        out_shape=[
            jax.ShapeDtypeStruct((N_PAD, fo), _f32),
            jax.ShapeDtypeStruct((N_PAD, fo), _f32),
            jax.ShapeDtypeStruct((2, N_PAD, half), _f32),
            jax.ShapeDtypeStruct((N_PAD, fo), _f32),
        ],
    )(x, wcat, bcat)


def _alpha_body(q_hbm, k_hbm, src_hbm, dst_hbm, alpha_out, wmax_out,
                src_l, dst_l, qb, kb, al, mx, sem1, sem2, *, d):
    cc = lax.axis_index("c")
    ss = lax.axis_index("s")
    w = ss * NC + cc
    inv = 1.0 / math.sqrt(float(d))
    pltpu.sync_copy(src_hbm.at[pl.ds(w * CH_A, CH_A)], src_l)
    pltpu.sync_copy(dst_hbm.at[pl.ds(w * CH_A, CH_A)], dst_l)

    lane_iota = lax.iota(_i32, L)

    def chunk_body(j, cmax):
        cp1 = pltpu.async_copy(q_hbm.at[dst_l.at[j]], qb, sem1)
        cp2 = pltpu.async_copy(k_hbm.at[src_l.at[j]], kb, sem2)
        cp1.wait()
        cp2.wait()

        def grp_body(g, m):
            rows = g * L + lane_iota
            col0 = jnp.zeros((L,), _i32)
            acc = (plsc.load_gather(qb, [rows, col0])
                   * plsc.load_gather(kb, [rows, col0]))
            for t in range(1, d):
                colt = jnp.full((L,), t, _i32)
                acc = acc + (plsc.load_gather(qb, [rows, colt])
                             * plsc.load_gather(kb, [rows, colt]))
            avec = acc * inv
            al[pl.ds(j * C + g * L, L)] = avec
            return jnp.maximum(m, avec)

        return lax.fori_loop(0, C // L, grp_body, cmax)

    m = lax.fori_loop(0, CH_A, chunk_body, jnp.full((L,), -1e30, _f32))
    mx[...] = m
    pltpu.sync_copy(al, alpha_out.at[pl.ds(w * EW_A, EW_A)])
    pltpu.sync_copy(mx, wmax_out.at[w])


def _sc_alpha(q, k, src2d, dst2d, d):
    mesh = plsc.VectorSubcoreMesh(core_axis_name="c", subcore_axis_name="s")
    f = pl.kernel(
        functools.partial(_alpha_body, d=d),
        compiler_params=pltpu.CompilerParams(use_tc_tiling_on_sc=False, needs_layout_passes=False),
        out_type=[
            jax.ShapeDtypeStruct((E_PAD,), _f32),
            jax.ShapeDtypeStruct((NW, L), _f32),
        ],
        mesh=mesh,
        scratch_types=[
            pltpu.VMEM((CH_A, C), _i32),
            pltpu.VMEM((CH_A, C), _i32),
            pltpu.VMEM((C, d), _f32),
            pltpu.VMEM((C, d), _f32),
            pltpu.VMEM((EW_A,), _f32),
            pltpu.VMEM((L,), _f32),
            pltpu.SemaphoreType.DMA,
            pltpu.SemaphoreType.DMA,
        ],
    )
    return f(q, k, src2d, dst2d)


def _agg_body(vflat_hbm, s_hbm, alpha_hbm, wmax_hbm, src_hbm, dst_hbm,
              h_out, den_out, src_l, dst_l, abuf, den, vbuf, stage,
              mxl, sem1, dfull_sh, agg_sh, *, fo):
    half = fo // 2
    cc = lax.axis_index("c")
    ss = lax.axis_index("s")

    pltpu.sync_copy(
        s_hbm.at[pl.ds(ss * RPS, RPS), pl.ds(cc * half, half)],
        agg_sh.at[pl.ds(ss * RPS, RPS)])

    lane_iota = lax.iota(_i32, L)
    pltpu.sync_copy(wmax_hbm, mxl)
    macc = mxl[0]
    for r in range(1, NW):
        macc = jnp.maximum(macc, mxl[r])
    for shift in (8, 4, 2, 1):
        abuf[pl.ds(0, L)] = macc
        rot = plsc.load_gather(abuf, [(lane_iota + shift) & (L - 1)])
        macc = jnp.maximum(macc, rot)
    gmax = macc

    pltpu.sync_copy(src_hbm.at[pl.ds(ss * CH_B2, CH_B2)], src_l)
    pltpu.sync_copy(dst_hbm.at[pl.ds(ss * CH_B2, CH_B2)], dst_l)

    def src_off_body(i, _):
        src_l[i // G2, pl.ds((i % G2) * L, L)] = (
            src_l[i // G2, pl.ds((i % G2) * L, L)] + cc * N_PAD)
        return 0

    lax.fori_loop(0, CH_B2 * G2, src_off_body, 0)

    def zero_body(i, _):
        den[pl.ds(i * L, L)] = jnp.zeros((L,), _f32)
        return 0

    lax.fori_loop(0, N_PAD // L, zero_body, 0)

    def den_grp(grp, _):
        pltpu.sync_copy(
            alpha_hbm.at[pl.ds(ss * EW_B + grp * AB, AB)], abuf)

        def den_chunk(jj, _2):
            j = grp * (AB // C2) + jj
            for t in range(G2):
                dstv = dst_l[j, pl.ds(t * L, L)]
                ex = jnp.exp(abuf[pl.ds(jj * C2 + t * L, L)] - gmax)
                plsc.addupdate_scatter(den, [dstv], ex)
            return 0

        lax.fori_loop(0, AB // C2, den_chunk, 0)
        return 0

    lax.fori_loop(0, EW_B // AB, den_grp, 0)

    pltpu.sync_copy(den, den_out.at[cc, ss])
    plsc.subcore_barrier()
    for r in range(NS):
        pltpu.sync_copy(den_out.at[cc, r, pl.ds(ss * RPS, RPS)], abuf)
        if r == 0:
            def cp_body(i, _):
                den[pl.ds(ss * RPS + i * L, L)] = abuf[pl.ds(i * L, L)]
                return 0
            lax.fori_loop(0, RPS // L, cp_body, 0)
        else:
            def add_body(i, _):
                den[pl.ds(ss * RPS + i * L, L)] = (
                    den[pl.ds(ss * RPS + i * L, L)] + abuf[pl.ds(i * L, L)])
                return 0
            lax.fori_loop(0, RPS // L, add_body, 0)
    pltpu.sync_copy(den.at[pl.ds(ss * RPS, RPS)], dfull_sh.at[pl.ds(ss * RPS, RPS)])
    plsc.subcore_barrier()
    pltpu.sync_copy(dfull_sh, den)

    def agg_grp(grp, _):
        pltpu.sync_copy(
            alpha_hbm.at[pl.ds(ss * EW_B + grp * AB, AB)], abuf)

        def agg_chunk(jj, _2):
            j = grp * (AB // C2) + jj
            cp = pltpu.async_copy(vflat_hbm.at[src_l.at[j]], vbuf, sem1)
            cp.wait()
            for t in range(G2):
                dstv = dst_l[j, pl.ds(t * L, L)]
                dv = plsc.load_gather(den, [dstv])
                ex = jnp.exp(abuf[pl.ds(jj * C2 + t * L, L)] - gmax)
                abuf[pl.ds(jj * C2 + t * L, L)] = ex / dv

            def edge_body(e, _3):
                wsc = plsc.load_gather(
                    abuf, [jnp.full((L,), jj * C2, _i32) + e])
                for t in range(half // L):
                    stage[e, pl.ds(t * L, L)] = vbuf[e, pl.ds(t * L, L)] * wsc
                return 0

            lax.fori_loop(0, C2, edge_body, 0)
            pltpu.sync_copy(stage, agg_sh.at[dst_l.at[j]], add=True)
            return 0

        lax.fori_loop(0, AB // C2, agg_chunk, 0)
        return 0

    lax.fori_loop(0, EW_B // AB, agg_grp, 0)
    plsc.subcore_barrier()

    pltpu.sync_copy(
        agg_sh.at[pl.ds(ss * RPS, RPS)],
        h_out.at[pl.ds(ss * RPS, RPS), pl.ds(cc * half, half)])


def _sc_agg(vflat, s, alpha, wmax, src2d_b, dst2d_b, fo):
    half = fo // 2
    mesh = plsc.VectorSubcoreMesh(core_axis_name="c", subcore_axis_name="s")
    f = pl.kernel(
        functools.partial(_agg_body, fo=fo),
        compiler_params=pltpu.CompilerParams(use_tc_tiling_on_sc=False, needs_layout_passes=False),
        out_type=[
            jax.ShapeDtypeStruct((N_PAD, fo), _f32),
            jax.ShapeDtypeStruct((NC, NS, N_PAD), _f32),
        ],
        mesh=mesh,
        scratch_types=[
            pltpu.VMEM((CH_B2, C2), _i32),
            pltpu.VMEM((CH_B2, C2), _i32),
            pltpu.VMEM((AB,), _f32),
            pltpu.VMEM((N_PAD,), _f32),
            pltpu.VMEM((C2, half), _f32),
            pltpu.VMEM((C2, half), _f32),
            pltpu.VMEM((NW, L), _f32),
            pltpu.SemaphoreType.DMA,
            pltpu.VMEM_SHARED((N_PAD,), _f32),
            pltpu.VMEM_SHARED((N_PAD, half), _f32),
        ],
    )
    h, _den = f(vflat, s, alpha, wmax, src2d_b, dst2d_b)
    return h


def _layer(x, wq, bq, wk, bk, wv, bv, ws, bs, src2d, dst2d, fo, elu):
    wcat = jnp.concatenate([wq, wk, wv, ws], axis=1)
    bcat = jnp.concatenate([bq, bk, bv, bs])[None, :]
    q, k, v2, s = _proj(x, wcat, bcat, fo, elu)
    alpha, wmax = _sc_alpha(q, k, src2d, dst2d, fo)
    vflat = v2.reshape(2 * N_PAD, fo // 2)
    h = _sc_agg(vflat, s, alpha, wmax,
                src2d.reshape(NCH2, C2), dst2d.reshape(NCH2, C2), fo)
    return h


def kernel(feat, edge, Wq1, bq1, Wk1, bk1, Wv1, bv1, Ws1, bs1, Wq2, bq2, Wk2, bk2, Wv2, bv2, Ws2, bs2, Wq3, bq3, Wk3, bk3, Wv3, bv3, Ws3, bs3, Wq4, bq4, Wk4, bk4, Wv4, bv4, Ws4, bs4):
    src = edge[0].astype(_i32)
    dst = edge[1].astype(_i32)
    pad_e = E_PAD - E
    src2d = jnp.concatenate([src, jnp.zeros((pad_e,), _i32)]).reshape(NCH, C)
    dst2d = jnp.concatenate([dst, jnp.full((pad_e,), N, _i32)]).reshape(NCH, C)
    x = jnp.concatenate([feat, jnp.zeros((N_PAD - N, feat.shape[1]), _f32)])

    h1 = _layer(x, Wq1, bq1, Wk1, bk1, Wv1, bv1, Ws1, bs1, src2d, dst2d, 256, False)
    h2 = _layer(h1, Wq2, bq2, Wk2, bk2, Wv2, bv2, Ws2, bs2, src2d, dst2d, 128, True)
    h3 = _layer(h2, Wq3, bq3, Wk3, bk3, Wv3, bv3, Ws3, bs3, src2d, dst2d, 256, False)
    h4 = _layer(h3, Wq4, bq4, Wk4, bk4, Wv4, bv4, Ws4, bs4, src2d, dst2d, 256, True)
    return (h2[:N], h4[:N])


# speedup vs baseline: 2.6955x; 1.5654x over previous
import functools
import math

import jax
import jax.numpy as jnp
from jax import lax
from jax.experimental import pallas as pl
from jax.experimental.pallas import tpu as pltpu
from jax.experimental.pallas import tpu_sc as plsc

N = 10000
E = 160000
N_PAD = 10240
E_PAD = 163840
NC, NS, L = 2, 16, 16
NW = NC * NS
RPS = N_PAD // NS

CA = 64
CHA = E_PAD // (NW * CA)
EW_A = CHA * CA

C2 = 32
CH_B2 = E_PAD // (NS * C2)
EW_B = CH_B2 * C2
G2 = C2 // L
AB = 640
ABC = AB // C2
NGRP = EW_B // AB

_f32 = jnp.float32
_i32 = jnp.int32


def _proj_body(x_ref, w_ref, b_ref, q_ref, k_ref, v_ref, s_ref, *, fo, elu):
    x = x_ref[...]
    if elu:
        x = jnp.where(x > 0, x, jnp.exp(x) - 1.0)
    acc = jnp.dot(x, w_ref[...], preferred_element_type=_f32) + b_ref[...]
    half = fo // 2
    q_ref[...] = acc[:, :fo]
    k_ref[...] = acc[:, fo:2 * fo]
    v_ref[0] = acc[:, 2 * fo:2 * fo + half]
    v_ref[1] = acc[:, 2 * fo + half:3 * fo]
    s_ref[...] = acc[:, 3 * fo:]


def _proj(x, wcat, bcat, fo, elu):
    fi = x.shape[1]
    half = fo // 2
    bm = 1024
    grid = (N_PAD // bm,)
    return pl.pallas_call(
        functools.partial(_proj_body, fo=fo, elu=elu),
        grid=grid,
        in_specs=[
            pl.BlockSpec((bm, fi), lambda i: (i, 0)),
            pl.BlockSpec((fi, 4 * fo), lambda i: (0, 0)),
            pl.BlockSpec((1, 4 * fo), lambda i: (0, 0)),
        ],
        out_specs=[
            pl.BlockSpec((bm, fo), lambda i: (i, 0)),
            pl.BlockSpec((bm, fo), lambda i: (i, 0)),
            pl.BlockSpec((2, bm, half), lambda i: (0, i, 0)),
            pl.BlockSpec((bm, fo), lambda i: (i, 0)),
        ],
        out_shape=[
            jax.ShapeDtypeStruct((N_PAD, fo), _f32),
            jax.ShapeDtypeStruct((N_PAD, fo), _f32),
            jax.ShapeDtypeStruct((2, N_PAD, half), _f32),
            jax.ShapeDtypeStruct((N_PAD, fo), _f32),
        ],
    )(x, wcat, bcat)


def _gath_body(q_hbm, k_hbm, src_hbm, dst_hbm, qg_out, kg_out,
               src_l, dst_l, qb0, qb1, kb0, kb1,
               semg0, semg1, semw0, semw1):
    cc = lax.axis_index("c")
    ss = lax.axis_index("s")
    w = ss * NC + cc
    pltpu.sync_copy(src_hbm.at[pl.ds(w * CHA, CHA)], src_l)
    pltpu.sync_copy(dst_hbm.at[pl.ds(w * CHA, CHA)], dst_l)

    qbs, kbs = (qb0, qb1), (kb0, kb1)
    semgs, semws = (semg0, semg1), (semw0, semw1)

    pltpu.async_copy(q_hbm.at[dst_l.at[0]], qb0, semg0)
    pltpu.async_copy(k_hbm.at[src_l.at[0]], kb0, semg0)

    def pair_body(jj, _):
        for par in range(2):
            j = jj * 2 + par
            qb, kb = qbs[par], kbs[par]
            semg, semw = semgs[par], semws[par]
            qb2, kb2 = qbs[1 - par], kbs[1 - par]
            pltpu.make_async_copy(q_hbm.at[dst_l.at[j]], qb, semg).wait()
            pltpu.make_async_copy(k_hbm.at[src_l.at[j]], kb, semg).wait()
            pltpu.async_copy(qb, qg_out.at[pl.ds(w * EW_A + j * CA, CA)],
                             semw)
            pltpu.async_copy(kb, kg_out.at[pl.ds(w * EW_A + j * CA, CA)],
                             semw)

            @pl.when(j + 1 < CHA)
            def _issue():
                @pl.when(j >= 1)
                def _wait_w():
                    off = w * EW_A + (j - 1) * CA
                    pltpu.make_async_copy(
                        qb2, qg_out.at[pl.ds(off, CA)], semws[1 - par]).wait()
                    pltpu.make_async_copy(
                        kb2, kg_out.at[pl.ds(off, CA)], semws[1 - par]).wait()
                pltpu.async_copy(q_hbm.at[dst_l.at[j + 1]], qb2,
                                 semgs[1 - par])
                pltpu.async_copy(k_hbm.at[src_l.at[j + 1]], kb2,
                                 semgs[1 - par])
        return 0

    lax.fori_loop(0, CHA // 2, pair_body, 0)
    for j in (CHA - 2, CHA - 1):
        off = w * EW_A + j * CA
        pltpu.make_async_copy(
            qbs[j % 2], qg_out.at[pl.ds(off, CA)], semws[j % 2]).wait()
        pltpu.make_async_copy(
            kbs[j % 2], kg_out.at[pl.ds(off, CA)], semws[j % 2]).wait()


def _sc_gather(q, k, src2d, dst2d, d):
    mesh = plsc.VectorSubcoreMesh(core_axis_name="c", subcore_axis_name="s")
    f = pl.kernel(
        _gath_body,
        compiler_params=pltpu.CompilerParams(
            use_tc_tiling_on_sc=False, needs_layout_passes=False),
        out_type=[
            jax.ShapeDtypeStruct((E_PAD, d), _f32),
            jax.ShapeDtypeStruct((E_PAD, d), _f32),
        ],
        mesh=mesh,
        scratch_types=[
            pltpu.VMEM((CHA, CA), _i32),
            pltpu.VMEM((CHA, CA), _i32),
            pltpu.VMEM((CA, d), _f32),
            pltpu.VMEM((CA, d), _f32),
            pltpu.VMEM((CA, d), _f32),
            pltpu.VMEM((CA, d), _f32),
            pltpu.SemaphoreType.DMA,
            pltpu.SemaphoreType.DMA,
            pltpu.SemaphoreType.DMA,
            pltpu.SemaphoreType.DMA,
        ],
    )
    return f(q, k, src2d, dst2d)


NB = 32
BE = E_PAD // NB
NBR = NB // L


def _tca_body(qg_ref, kg_ref, a_ref, m_ref, *, inv):
    s = jnp.sum(qg_ref[...] * kg_ref[...], axis=1) * inv
    a_ref[...] = s
    m_ref[...] = jnp.broadcast_to(jnp.max(s), (1, 8, 128))


def _tc_alpha(qg, kg, d):
    inv = 1.0 / math.sqrt(float(d))
    return pl.pallas_call(
        functools.partial(_tca_body, inv=inv),
        grid=(NB,),
        in_specs=[
            pl.BlockSpec((BE, d), lambda i: (i, 0)),
            pl.BlockSpec((BE, d), lambda i: (i, 0)),
        ],
        out_specs=[
            pl.BlockSpec((BE,), lambda i: (i,)),
            pl.BlockSpec((1, 8, 128), lambda i: (i, 0, 0)),
        ],
        out_shape=[
            jax.ShapeDtypeStruct((E_PAD,), _f32),
            jax.ShapeDtypeStruct((NB, 8, 128), _f32),
        ],
    )(qg, kg)


def _agg_body(vflat_hbm, s_hbm, alpha_hbm, wmax_hbm, src_hbm, dst_hbm,
              h_out, den_out, src_l, dst_l, abuf, den, vb0, vb1, st0, st1,
              mxl, semg0, semg1, sems0, sems1, dfull_sh, agg_sh, *, fo):
    half = fo // 2
    cc = lax.axis_index("c")
    ss = lax.axis_index("s")

    pltpu.sync_copy(
        s_hbm.at[pl.ds(ss * RPS, RPS), pl.ds(cc * half, half)],
        agg_sh.at[pl.ds(ss * RPS, RPS)])

    lane_iota = lax.iota(_i32, L)
    pltpu.sync_copy(wmax_hbm, mxl)
    macc = mxl[0]
    for r in range(1, NBR):
        macc = jnp.maximum(macc, mxl[r])
    for shift in (8, 4, 2, 1):
        abuf[pl.ds(0, L)] = macc
        rot = plsc.load_gather(abuf, [(lane_iota + shift) & (L - 1)])
        macc = jnp.maximum(macc, rot)
    gmax = macc

    pltpu.sync_copy(src_hbm.at[pl.ds(ss * CH_B2, CH_B2)], src_l)
    pltpu.sync_copy(dst_hbm.at[pl.ds(ss * CH_B2, CH_B2)], dst_l)

    def src_off_body(i, _):
        src_l[i // G2, pl.ds((i % G2) * L, L)] = (
            src_l[i // G2, pl.ds((i % G2) * L, L)] + cc * N_PAD)
        return 0

    lax.fori_loop(0, CH_B2 * G2, src_off_body, 0)

    def zero_body(i, _):
        den[pl.ds(i * L, L)] = jnp.zeros((L,), _f32)
        return 0

    lax.fori_loop(0, N_PAD // L, zero_body, 0)

    def den_grp(grp, _):
        pltpu.sync_copy(
            alpha_hbm.at[pl.ds(ss * EW_B + grp * AB, AB)], abuf)

        def den_sub(i, _2):
            jloc = i // G2
            dstv = dst_l[grp * ABC + jloc, pl.ds((i % G2) * L, L)]
            ex = jnp.exp(abuf[pl.ds(i * L, L)] - gmax)
            plsc.addupdate_scatter(den, [dstv], ex)
            return 0

        lax.fori_loop(0, AB // L, den_sub, 0)
        return 0

    lax.fori_loop(0, NGRP, den_grp, 0)

    pltpu.sync_copy(den, den_out.at[cc, ss])
    plsc.subcore_barrier()
    for r in range(NS):
        pltpu.sync_copy(den_out.at[cc, r, pl.ds(ss * RPS, RPS)], abuf)
        if r == 0:
            def cp_body(i, _):
                den[pl.ds(ss * RPS + i * L, L)] = abuf[pl.ds(i * L, L)]
                return 0
            lax.fori_loop(0, RPS // L, cp_body, 0)
        else:
            def add_body(i, _):
                den[pl.ds(ss * RPS + i * L, L)] = (
                    den[pl.ds(ss * RPS + i * L, L)] + abuf[pl.ds(i * L, L)])
                return 0
            lax.fori_loop(0, RPS // L, add_body, 0)
    pltpu.sync_copy(den.at[pl.ds(ss * RPS, RPS)],
                    dfull_sh.at[pl.ds(ss * RPS, RPS)])
    plsc.subcore_barrier()
    pltpu.sync_copy(dfull_sh, den)

    vbs, sts = (vb0, vb1), (st0, st1)
    semgs, semss = (semg0, semg1), (sems0, sems1)

    pltpu.async_copy(vflat_hbm.at[src_l.at[0]], vb0, semg0)

    def agg_grp(grp, _):
        pltpu.sync_copy(
            alpha_hbm.at[pl.ds(ss * EW_B + grp * AB, AB)], abuf)

        def chunk_pair(jj, _2):
            for par in range(2):
                jloc = jj * 2 + par
                j = grp * ABC + jloc
                vb, st = vbs[par], sts[par]
                semg, sems = semgs[par], semss[par]
                pltpu.make_async_copy(
                    vflat_hbm.at[src_l.at[j]], vb, semg).wait()

                @pl.when(j + 1 < CH_B2)
                def _issue():
                    pltpu.async_copy(vflat_hbm.at[src_l.at[j + 1]],
                                     vbs[1 - par], semgs[1 - par])

                for t in range(G2):
                    dstv = dst_l[j, pl.ds(t * L, L)]
                    dv = plsc.load_gather(den, [dstv])
                    ex = jnp.exp(abuf[pl.ds(jloc * C2 + t * L, L)] - gmax)
                    abuf[pl.ds(jloc * C2 + t * L, L)] = ex / dv

                @pl.when(j >= 2)
                def _wait_sc():
                    pltpu.make_async_copy(
                        st, agg_sh.at[dst_l.at[j]], sems).wait()

                def edge_body(e, _3):
                    wsc = plsc.load_gather(
                        abuf, [jnp.full((L,), jloc * C2, _i32) + e])
                    for t in range(half // L):
                        st[e, pl.ds(t * L, L)] = vb[e, pl.ds(t * L, L)] * wsc
                    return 0

                lax.fori_loop(0, C2, edge_body, 0)
                pltpu.async_copy(st, agg_sh.at[dst_l.at[j]], sems, add=True)
            return 0

        lax.fori_loop(0, ABC // 2, chunk_pair, 0)
        return 0

    lax.fori_loop(0, NGRP, agg_grp, 0)
    pltpu.make_async_copy(st0, agg_sh.at[dst_l.at[CH_B2 - 2]], sems0).wait()
    pltpu.make_async_copy(st1, agg_sh.at[dst_l.at[CH_B2 - 1]], sems1).wait()
    plsc.subcore_barrier()

    pltpu.sync_copy(
        agg_sh.at[pl.ds(ss * RPS, RPS)],
        h_out.at[pl.ds(ss * RPS, RPS), pl.ds(cc * half, half)])


def _sc_agg(vflat, s, alpha, wmax, src2d_b, dst2d_b, fo):
    half = fo // 2
    mesh = plsc.VectorSubcoreMesh(core_axis_name="c", subcore_axis_name="s")
    f = pl.kernel(
        functools.partial(_agg_body, fo=fo),
        compiler_params=pltpu.CompilerParams(
            use_tc_tiling_on_sc=False, needs_layout_passes=False),
        out_type=[
            jax.ShapeDtypeStruct((N_PAD, fo), _f32),
            jax.ShapeDtypeStruct((NC, NS, N_PAD), _f32),
        ],
        mesh=mesh,
        scratch_types=[
            pltpu.VMEM((CH_B2, C2), _i32),
            pltpu.VMEM((CH_B2, C2), _i32),
            pltpu.VMEM((AB,), _f32),
            pltpu.VMEM((N_PAD,), _f32),
            pltpu.VMEM((C2, half), _f32),
            pltpu.VMEM((C2, half), _f32),
            pltpu.VMEM((C2, half), _f32),
            pltpu.VMEM((C2, half), _f32),
            pltpu.VMEM((NBR, L), _f32),
            pltpu.SemaphoreType.DMA,
            pltpu.SemaphoreType.DMA,
            pltpu.SemaphoreType.DMA,
            pltpu.SemaphoreType.DMA,
            pltpu.VMEM_SHARED((N_PAD,), _f32),
            pltpu.VMEM_SHARED((N_PAD, half), _f32),
        ],
    )
    h, _den = f(vflat, s, alpha, wmax, src2d_b, dst2d_b)
    return h


def _layer(x, wq, bq, wk, bk, wv, bv, ws, bs, srcA, dstA, srcB, dstB,
           fo, elu):
    wcat = jnp.concatenate([wq, wk, wv, ws], axis=1)
    bcat = jnp.concatenate([bq, bk, bv, bs])[None, :]
    q, k, v2, s = _proj(x, wcat, bcat, fo, elu)
    qg, kg = _sc_gather(q, k, srcA, dstA, fo)
    alpha, bmax = _tc_alpha(qg, kg, fo)
    vflat = v2.reshape(2 * N_PAD, fo // 2)
    h = _sc_agg(vflat, s, alpha, bmax[:, 0, 0].reshape(NBR, L),
                srcB, dstB, fo)
    return h


def kernel(feat, edge, Wq1, bq1, Wk1, bk1, Wv1, bv1, Ws1, bs1, Wq2, bq2, Wk2, bk2, Wv2, bv2, Ws2, bs2, Wq3, bq3, Wk3, bk3, Wv3, bv3, Ws3, bs3, Wq4, bq4, Wk4, bk4, Wv4, bv4, Ws4, bs4):
    src = edge[0].astype(_i32)
    dst = edge[1].astype(_i32)
    pad_e = E_PAD - E
    src_p = jnp.concatenate([src, jnp.zeros((pad_e,), _i32)])
    dst_p = jnp.concatenate([dst, jnp.full((pad_e,), N, _i32)])
    srcA = src_p.reshape(NW * CHA, CA)
    dstA = dst_p.reshape(NW * CHA, CA)
    srcB = src_p.reshape(NS * CH_B2, C2)
    dstB = dst_p.reshape(NS * CH_B2, C2)
    x = jnp.concatenate([feat, jnp.zeros((N_PAD - N, feat.shape[1]), _f32)])

    h1 = _layer(x, Wq1, bq1, Wk1, bk1, Wv1, bv1, Ws1, bs1,
                srcA, dstA, srcB, dstB, 256, False)
    h2 = _layer(h1, Wq2, bq2, Wk2, bk2, Wv2, bv2, Ws2, bs2,
                srcA, dstA, srcB, dstB, 128, True)
    h3 = _layer(h2, Wq3, bq3, Wk3, bk3, Wv3, bv3, Ws3, bs3,
                srcA, dstA, srcB, dstB, 256, False)
    h4 = _layer(h3, Wq4, bq4, Wk4, bk4, Wv4, bv4, Ws4, bs4,
                srcA, dstA, srcB, dstB, 256, True)
    return (h2[:N], h4[:N])
